# speedup vs baseline: 131.1387x; 131.1387x over previous
import numpy as np
import jax
import jax.numpy as jnp
from jax.experimental import pallas as pl
from jax.experimental.pallas import tpu as pltpu

Z_CH = 16
SPK = 8
CH0 = 32
CH1 = 16
CH2 = 8
T0 = 8
T1 = 16
T2 = 32
N_FFT = 16
HOP = 4
SUBBANDS = 4
F_BINS = N_FFT // 2 + 1
FRAMES = T2 + 1
TRIM = (N_FFT // 2) // HOP
T_MB = HOP * (FRAMES - 1)
PQ_TAPS = 62
PQ_NPOLY = (PQ_TAPS + SUBBANDS) // SUBBANDS
PQ_SHIFT = (PQ_TAPS // 2 - (SUBBANDS - 1)) // SUBBANDS
RES_KERNELS = (3, 5)
RES_DILS = ((1, 3, 5), (1, 3, 5))
SB_W = 2 * F_BINS * FRAMES
RE_W = F_BINS * FRAMES


def _conv_dense(w, b, c_in, c_out, t_in, t_out, dil, pad):
    ksz = w.shape[0]
    sel = np.zeros((ksz, t_in, t_out), np.float32)
    for k in range(ksz):
        for to in range(t_out):
            ti = to + k * dil - pad
            if 0 <= ti < t_in:
                sel[k, ti, to] = 1.0
    m = jnp.einsum('koc,ktu->ctou', w, sel).reshape(c_in * t_in, c_out * t_out)
    bias = jnp.broadcast_to(b.reshape(c_out, 1), (c_out, t_out)).reshape(1, c_out * t_out)
    return m, bias


def _convT_dense(w, b, c_in, c_out, t_in):
    t_out = 2 * t_in
    sel = np.zeros((4, t_in, t_out), np.float32)
    for m_ in range(t_in):
        if m_ - 1 >= 0:
            sel[0, m_ - 1, 2 * m_] = 1.0
        sel[2, m_, 2 * m_] = 1.0
        sel[1, m_, 2 * m_ + 1] = 1.0
        if m_ + 1 < t_in:
            sel[3, m_ + 1, 2 * m_ + 1] = 1.0
    m = jnp.einsum('koc,ktu->ctou', w, sel).reshape(c_in * t_in, c_out * t_out)
    bias = jnp.broadcast_to(b.reshape(c_out, 1), (c_out, t_out)).reshape(1, c_out * t_out)
    return m, bias


def _post_dense(w, b):
    sel = np.zeros((7, T2, FRAMES), np.float32)
    for k in range(7):
        for to in range(FRAMES):
            u = to + k
            if u == 3:
                sel[k, 1, to] += 1.0
            elif 4 <= u < 36:
                sel[k, u - 4, to] += 1.0
    c_out = w.shape[1]
    m = jnp.einsum('koc,ktu->ctou', w, sel).reshape(CH2 * T2, c_out * FRAMES)
    bias = jnp.broadcast_to(b.reshape(c_out, 1), (c_out, FRAMES)).reshape(1, c_out * FRAMES)
    return m, bias


def _rb_dense(w, b, ks, dils, ch, t):
    n = len(dils)
    mats, biases = [], []
    for l, d in enumerate(dils):
        m1, b1 = _conv_dense(w[l * ks:(l + 1) * ks], b[l], ch, ch, t, t,
                             d, (ks - 1) * d // 2)
        m2, b2 = _conv_dense(w[(n + l) * ks:(n + l + 1) * ks], b[n + l],
                             ch, ch, t, t, 1, (ks - 1) // 2)
        mats += [m1, m2]
        biases += [b1, b2]
    return mats, biases


def _istft_tail_mats(cre, cim, inv_env):
    t_sel = np.zeros((HOP, FRAMES, T2), np.float32)
    for a in range(HOP):
        for m_ in range(T2):
            j = m_ + TRIM - a
            if 0 <= j < FRAMES:
                t_sel[a, j, m_] = 1.0
    cre4 = cre.reshape(HOP, HOP, F_BINS)
    cim4 = cim.reshape(HOP, HOP, F_BINS)
    a_re = jnp.einsum('ajm,abf,bm->fjmb', t_sel, cre4, inv_env).reshape(RE_W, T_MB)
    a_im = jnp.einsum('ajm,abf,bm->fjmb', t_sel, cim4, inv_env).reshape(RE_W, T_MB)
    return a_re, a_im


def _pqmf_dense(pqmf_wp):
    u_sel = np.zeros((PQ_NPOLY, T_MB, T_MB), np.float32)
    for i in range(PQ_NPOLY):
        for q in range(T_MB):
            qi = q + i - PQ_SHIFT
            if 0 <= qi < T_MB:
                u_sel[i, qi, q] = 1.0
    return jnp.einsum('iuq,irs->suqr', u_sel, pqmf_wp).reshape(
        SUBBANDS * T_MB, SUBBANDS * T_MB)


def _decoder_block_kernel(z_ref, spk_ref, mz, mspk, bpre, mu0, bu0, m0, b0,
                          mu1, bu1, m1, b1, mpost, bpost, are, aim, mpq,
                          wav_ref, ymb_ref):
    f32 = jnp.float32

    def dot(a, bm):
        return jnp.dot(a, bm, preferred_element_type=f32)

    def lk(v, s):
        return jnp.where(v >= 0, v, s * v)

    def rb_stage(x, mref, bref):
        acc = None
        for rbi in range(2):
            h = x
            for l in range(3):
                j = rbi * 6 + l * 2
                xt = dot(lk(h, 0.1), mref[j]) + bref[j:j + 1]
                h = dot(lk(xt, 0.1), mref[j + 1]) + bref[j + 1:j + 2] + h
            acc = h if acc is None else acc + h
        return acc * 0.5

    x = dot(z_ref[...], mz[...]) + dot(spk_ref[...], mspk[...]) + bpre[...]
    x = dot(lk(x, 0.1), mu0[...]) + bu0[...]
    x = rb_stage(x, m0, b0)
    x = dot(lk(x, 0.1), mu1[...]) + bu1[...]
    x = rb_stage(x, m1, b1)
    p = dot(lk(x, 0.01), mpost[...]) + bpost[...]
    parts = []
    for s in range(SUBBANDS):
        lm = p[:, s * SB_W:s * SB_W + RE_W]
        pin = p[:, s * SB_W + RE_W:s * SB_W + 2 * RE_W]
        mag = jnp.exp(lm)
        ph = jnp.pi * jnp.sin(pin)
        ys = dot(mag * jnp.cos(ph), are[...]) + dot(mag * jnp.sin(ph), aim[...])
        ymb_ref[:, s, :] = ys
        parts.append(ys)
    wav_ref[:, 0, :] = dot(jnp.concatenate(parts, axis=1), mpq[...])


def kernel(z, spk, pre_w, pre_b, cond_w, cond_b, up0_w, up0_b, up1_w, up1_b,
           rb00_w, rb00_b, rb01_w, rb01_b, rb10_w, rb10_b, rb11_w, rb11_b,
           post_w, post_b, cre, cim, inv_env, pqmf_wp):
    bn = z.shape[0]

    mz, bz = _conv_dense(pre_w, pre_b, Z_CH, CH0, T0, T0, 1, 3)
    mspk = jnp.einsum('os,t->sot', cond_w, np.ones(T0, np.float32)).reshape(SPK, CH0 * T0)
    bcond = jnp.broadcast_to(cond_b.reshape(CH0, 1), (CH0, T0)).reshape(1, CH0 * T0)
    bpre = bz + bcond
    mu0, bu0 = _convT_dense(up0_w, up0_b, CH0, CH1, T0)
    mu1, bu1 = _convT_dense(up1_w, up1_b, CH1, CH2, T1)
    mats0, biases0 = [], []
    for w, b, ks, dl in ((rb00_w, rb00_b, RES_KERNELS[0], RES_DILS[0]),
                         (rb01_w, rb01_b, RES_KERNELS[1], RES_DILS[1])):
        ms, bs = _rb_dense(w, b, ks, dl, CH1, T1)
        mats0 += ms
        biases0 += bs
    mats1, biases1 = [], []
    for w, b, ks, dl in ((rb10_w, rb10_b, RES_KERNELS[0], RES_DILS[0]),
                         (rb11_w, rb11_b, RES_KERNELS[1], RES_DILS[1])):
        ms, bs = _rb_dense(w, b, ks, dl, CH2, T2)
        mats1 += ms
        biases1 += bs
    m0 = jnp.stack(mats0)
    b0 = jnp.concatenate(biases0, axis=0)
    m1 = jnp.stack(mats1)
    b1 = jnp.concatenate(biases1, axis=0)
    mpost, bpost = _post_dense(post_w, post_b)
    are, aim = _istft_tail_mats(cre, cim, inv_env)
    mpq = _pqmf_dense(pqmf_wp)

    bb = 256
    while bn % bb:
        bb //= 2
    z2 = z.reshape(bn, Z_CH * T0)
    s2 = spk.reshape(bn, SPK)

    weights = [mz, mspk, bpre, mu0, bu0, m0, b0, mu1, bu1, m1, b1,
               mpost, bpost, are, aim, mpq]

    def rep_spec(a):
        nd = a.ndim
        return pl.BlockSpec(tuple(a.shape), lambda i, _n=nd: (0,) * _n)

    in_specs = ([pl.BlockSpec((bb, Z_CH * T0), lambda i: (i, 0)),
                 pl.BlockSpec((bb, SPK), lambda i: (i, 0))]
                + [rep_spec(a) for a in weights])
    out_specs = [pl.BlockSpec((bb, 1, SUBBANDS * T_MB), lambda i: (i, 0, 0)),
                 pl.BlockSpec((bb, SUBBANDS, T_MB), lambda i: (i, 0, 0))]
    out_shape = (jax.ShapeDtypeStruct((bn, 1, SUBBANDS * T_MB), jnp.float32),
                 jax.ShapeDtypeStruct((bn, SUBBANDS, T_MB), jnp.float32))
    wav, ymb = pl.pallas_call(
        _decoder_block_kernel,
        grid=(bn // bb,),
        in_specs=in_specs,
        out_specs=out_specs,
        out_shape=out_shape,
        compiler_params=pltpu.CompilerParams(dimension_semantics=("parallel",)),
    )(z2, s2, *weights)
    return wav, ymb


# speedup vs baseline: 292.1684x; 2.2279x over previous
import numpy as np
import jax
import jax.numpy as jnp
from jax.experimental import pallas as pl
from jax.experimental.pallas import tpu as pltpu

Z_CH = 16
SPK = 8
CH0 = 32
CH1 = 16
CH2 = 8
T0 = 8
T1 = 16
T2 = 32
N_FFT = 16
HOP = 4
SUBBANDS = 4
F_BINS = N_FFT // 2 + 1
FRAMES = T2 + 1
TRIM = (N_FFT // 2) // HOP
T_MB = HOP * (FRAMES - 1)
PQ_TAPS = 62
PQ_NPOLY = (PQ_TAPS + SUBBANDS) // SUBBANDS
PQ_SHIFT = (PQ_TAPS // 2 - (SUBBANDS - 1)) // SUBBANDS
RES_KERNELS = (3, 5)
RES_DILS = ((1, 3, 5), (1, 3, 5))
SB_W = 2 * F_BINS * FRAMES
RE_W = F_BINS * FRAMES


def _fit_trig_coeffs():
    a = np.pi * 1.02
    x = a * np.cos(np.linspace(0.0, np.pi, 4001))
    x = x[np.abs(x) > 1e-9]
    u = x * x
    sin_c = np.polyfit(u, np.sin(x) / x, 5)
    cos_c = np.polyfit(u, np.cos(x), 6)
    return tuple(float(c) for c in sin_c), tuple(float(c) for c in cos_c)


_SIN_C, _COS_C = _fit_trig_coeffs()
_TWO_PI = 2.0 * np.pi
_INV_TWO_PI = 1.0 / _TWO_PI


def _sin_poly(x):
    x2 = x * x
    p = _SIN_C[0]
    for c in _SIN_C[1:]:
        p = p * x2 + c
    return x * p


def _cos_poly(x):
    x2 = x * x
    p = _COS_C[0]
    for c in _COS_C[1:]:
        p = p * x2 + c
    return p


def _sin_any(x):
    n = jnp.round(x * _INV_TWO_PI)
    return _sin_poly(x - n * _TWO_PI)


def _conv_dense(w, b, c_in, c_out, t_in, t_out, dil, pad):
    ksz = w.shape[0]
    sel = np.zeros((ksz, t_in, t_out), np.float32)
    for k in range(ksz):
        for to in range(t_out):
            ti = to + k * dil - pad
            if 0 <= ti < t_in:
                sel[k, ti, to] = 1.0
    m = jnp.einsum('koc,ktu->ctou', w, sel).reshape(c_in * t_in, c_out * t_out)
    bias = jnp.broadcast_to(b.reshape(c_out, 1), (c_out, t_out)).reshape(1, c_out * t_out)
    return m, bias


def _convT_dense(w, b, c_in, c_out, t_in):
    t_out = 2 * t_in
    sel = np.zeros((4, t_in, t_out), np.float32)
    for m_ in range(t_in):
        if m_ - 1 >= 0:
            sel[0, m_ - 1, 2 * m_] = 1.0
        sel[2, m_, 2 * m_] = 1.0
        sel[1, m_, 2 * m_ + 1] = 1.0
        if m_ + 1 < t_in:
            sel[3, m_ + 1, 2 * m_ + 1] = 1.0
    m = jnp.einsum('koc,ktu->ctou', w, sel).reshape(c_in * t_in, c_out * t_out)
    bias = jnp.broadcast_to(b.reshape(c_out, 1), (c_out, t_out)).reshape(1, c_out * t_out)
    return m, bias


def _post_dense(w, b):
    sel = np.zeros((7, T2, FRAMES), np.float32)
    for k in range(7):
        for to in range(FRAMES):
            u = to + k
            if u == 3:
                sel[k, 1, to] += 1.0
            elif 4 <= u < 36:
                sel[k, u - 4, to] += 1.0
    c_out = w.shape[1]
    m = jnp.einsum('koc,ktu->ctou', w, sel).reshape(CH2 * T2, c_out * FRAMES)
    bias = jnp.broadcast_to(b.reshape(c_out, 1), (c_out, FRAMES)).reshape(1, c_out * FRAMES)
    return m, bias


def _rb_dense(w, b, ks, dils, ch, t):
    n = len(dils)
    mats, biases = [], []
    for l, d in enumerate(dils):
        m1, b1 = _conv_dense(w[l * ks:(l + 1) * ks], b[l], ch, ch, t, t,
                             d, (ks - 1) * d // 2)
        m2, b2 = _conv_dense(w[(n + l) * ks:(n + l + 1) * ks], b[n + l],
                             ch, ch, t, t, 1, (ks - 1) // 2)
        mats += [m1, m2]
        biases += [b1, b2]
    return mats, biases


def _istft_tail_mats(cre, cim, inv_env):
    t_sel = np.zeros((HOP, FRAMES, T2), np.float32)
    for a in range(HOP):
        for m_ in range(T2):
            j = m_ + TRIM - a
            if 0 <= j < FRAMES:
                t_sel[a, j, m_] = 1.0
    cre4 = cre.reshape(HOP, HOP, F_BINS)
    cim4 = cim.reshape(HOP, HOP, F_BINS)
    a_re = jnp.einsum('ajm,abf,bm->fjmb', t_sel, cre4, inv_env).reshape(RE_W, T_MB)
    a_im = jnp.einsum('ajm,abf,bm->fjmb', t_sel, cim4, inv_env).reshape(RE_W, T_MB)
    return a_re, a_im


def _pqmf_dense(pqmf_wp):
    u_sel = np.zeros((PQ_NPOLY, T_MB, T_MB), np.float32)
    for i in range(PQ_NPOLY):
        for q in range(T_MB):
            qi = q + i - PQ_SHIFT
            if 0 <= qi < T_MB:
                u_sel[i, qi, q] = 1.0
    return jnp.einsum('iuq,irs->suqr', u_sel, pqmf_wp).reshape(
        SUBBANDS * T_MB, SUBBANDS * T_MB)


def _decoder_block_kernel(z_ref, spk_ref, mz, mspk, bpre, mu0, bu0, m0, b0,
                          mu1, bu1, m1, b1, mpost, bpost, are, aim, mpq,
                          wav_ref, ymb_ref):
    f32 = jnp.float32

    def dot(a, bm):
        return jnp.dot(a, bm, preferred_element_type=f32)

    def lk(v, s):
        return jnp.maximum(v, s * v)

    def rb_stage(x, mref, bref):
        acc = None
        for rbi in range(2):
            h = x
            for l in range(3):
                j = rbi * 6 + l * 2
                xt = dot(lk(h, 0.1), mref[j]) + bref[j:j + 1]
                h = dot(lk(xt, 0.1), mref[j + 1]) + bref[j + 1:j + 2] + h
            acc = h if acc is None else acc + h
        return acc * 0.5

    x = dot(z_ref[...], mz[...]) + dot(spk_ref[...], mspk[...]) + bpre[...]
    x = dot(lk(x, 0.1), mu0[...]) + bu0[...]
    x = rb_stage(x, m0, b0)
    x = dot(lk(x, 0.1), mu1[...]) + bu1[...]
    x = rb_stage(x, m1, b1)
    p = dot(lk(x, 0.01), mpost[...]) + bpost[...]
    parts = []
    for s in range(SUBBANDS):
        lm = p[:, s * SB_W:s * SB_W + RE_W]
        pin = p[:, s * SB_W + RE_W:s * SB_W + 2 * RE_W]
        mag = jnp.exp(lm)
        ph = jnp.pi * _sin_any(pin)
        ys = dot(mag * _cos_poly(ph), are[...]) + dot(mag * _sin_poly(ph), aim[...])
        ymb_ref[:, s, :] = ys
        parts.append(ys)
    wav_ref[:, 0, :] = dot(jnp.concatenate(parts, axis=1), mpq[...])


def kernel(z, spk, pre_w, pre_b, cond_w, cond_b, up0_w, up0_b, up1_w, up1_b,
           rb00_w, rb00_b, rb01_w, rb01_b, rb10_w, rb10_b, rb11_w, rb11_b,
           post_w, post_b, cre, cim, inv_env, pqmf_wp):
    bn = z.shape[0]

    mz, bz = _conv_dense(pre_w, pre_b, Z_CH, CH0, T0, T0, 1, 3)
    mspk = jnp.einsum('os,t->sot', cond_w, np.ones(T0, np.float32)).reshape(SPK, CH0 * T0)
    bcond = jnp.broadcast_to(cond_b.reshape(CH0, 1), (CH0, T0)).reshape(1, CH0 * T0)
    bpre = bz + bcond
    mu0, bu0 = _convT_dense(up0_w, up0_b, CH0, CH1, T0)
    mu1, bu1 = _convT_dense(up1_w, up1_b, CH1, CH2, T1)
    mats0, biases0 = [], []
    for w, b, ks, dl in ((rb00_w, rb00_b, RES_KERNELS[0], RES_DILS[0]),
                         (rb01_w, rb01_b, RES_KERNELS[1], RES_DILS[1])):
        ms, bs = _rb_dense(w, b, ks, dl, CH1, T1)
        mats0 += ms
        biases0 += bs
    mats1, biases1 = [], []
    for w, b, ks, dl in ((rb10_w, rb10_b, RES_KERNELS[0], RES_DILS[0]),
                         (rb11_w, rb11_b, RES_KERNELS[1], RES_DILS[1])):
        ms, bs = _rb_dense(w, b, ks, dl, CH2, T2)
        mats1 += ms
        biases1 += bs
    m0 = jnp.stack(mats0)
    b0 = jnp.concatenate(biases0, axis=0)
    m1 = jnp.stack(mats1)
    b1 = jnp.concatenate(biases1, axis=0)
    mpost, bpost = _post_dense(post_w, post_b)
    are, aim = _istft_tail_mats(cre, cim, inv_env)
    mpq = _pqmf_dense(pqmf_wp)

    bb = 256
    while bn % bb:
        bb //= 2
    z2 = z.reshape(bn, Z_CH * T0)
    s2 = spk.reshape(bn, SPK)

    weights = [mz, mspk, bpre, mu0, bu0, m0, b0, mu1, bu1, m1, b1,
               mpost, bpost, are, aim, mpq]

    def rep_spec(a):
        nd = a.ndim
        return pl.BlockSpec(tuple(a.shape), lambda i, _n=nd: (0,) * _n)

    in_specs = ([pl.BlockSpec((bb, Z_CH * T0), lambda i: (i, 0)),
                 pl.BlockSpec((bb, SPK), lambda i: (i, 0))]
                + [rep_spec(a) for a in weights])
    out_specs = [pl.BlockSpec((bb, 1, SUBBANDS * T_MB), lambda i: (i, 0, 0)),
                 pl.BlockSpec((bb, SUBBANDS, T_MB), lambda i: (i, 0, 0))]
    out_shape = (jax.ShapeDtypeStruct((bn, 1, SUBBANDS * T_MB), jnp.float32),
                 jax.ShapeDtypeStruct((bn, SUBBANDS, T_MB), jnp.float32))
    wav, ymb = pl.pallas_call(
        _decoder_block_kernel,
        grid=(bn // bb,),
        in_specs=in_specs,
        out_specs=out_specs,
        out_shape=out_shape,
        compiler_params=pltpu.CompilerParams(dimension_semantics=("parallel",)),
    )(z2, s2, *weights)
    return wav, ymb
